# baseline (speedup 1.0000x reference)
import jax
import jax.numpy as jnp
from jax.experimental import pallas as pl
from jax.experimental.pallas import tpu as pltpu

_DN = (((1,), (0,)), ((), ()))


def _sage_kernel(adj_ref, x_ref, w_ref, out_ref, y2_ref, *, bm):
    i = pl.program_id(0)
    f_in = x_ref.shape[1]

    @pl.when(i == 0)
    def _build_y2():
        w2 = w_ref[pl.ds(f_in, f_in), :]
        y2_ref[...] = jax.lax.dot_general(
            x_ref[...], w2, _DN, preferred_element_type=jnp.float32,
            precision=jax.lax.Precision.DEFAULT)

    w1 = w_ref[pl.ds(0, f_in), :]
    x_band = x_ref[pl.ds(i * bm, bm), :]
    self_term = jax.lax.dot_general(
        x_band, w1, _DN, preferred_element_type=jnp.float32,
        precision=jax.lax.Precision.DEFAULT)
    out_ref[...] = self_term + jax.lax.dot_general(
        adj_ref[...], y2_ref[...], _DN, preferred_element_type=jnp.float32,
        precision=jax.lax.Precision.DEFAULT)


def kernel(input, adj, weight):
    n, f_in = input.shape
    f_out = weight.shape[1]
    bm = min(n, 400)

    import functools
    body = functools.partial(_sage_kernel, bm=bm)
    out = pl.pallas_call(
        body,
        grid=(n // bm,),
        in_specs=[
            pl.BlockSpec((bm, n), lambda i: (i, 0)),
            pl.BlockSpec((n, f_in), lambda i: (0, 0)),
            pl.BlockSpec((2 * f_in, f_out), lambda i: (0, 0)),
        ],
        out_specs=pl.BlockSpec((bm, f_out), lambda i: (i, 0)),
        out_shape=jax.ShapeDtypeStruct((n, f_out), jnp.float32),
        scratch_shapes=[pltpu.VMEM((n, f_out), jnp.float32)],
        compiler_params=pltpu.CompilerParams(
            dimension_semantics=("arbitrary",)),
    )(adj, input, weight)
    return out
